# baseline (speedup 1.0000x reference)
import functools

import jax
import jax.numpy as jnp
from jax import lax
from jax.experimental import pallas as pl
from jax.experimental.pallas import tpu as pltpu
from jax.experimental.pallas import tpu_sc as plsc

N = 10000
D = 2000
E = 40000
EP = E + N
DP = 2048

NC = 2
NS = 16
NW = NC * NS
NPT = 320
OFF_SL = 336
NP_PAD = 10240
OFFP = 10264
CC = 2048
SRC_CHUNKS = 20
SRC_CAP = SRC_CHUNKS * CC
W_CAP = 40368
EPP = 93184
GB = 4
NEG = -3.4e38

MB = 512
AW = 128
NPO = 10240


def _sget(ref, i):
    return ref[pl.ds(i, 16)][0]


def _mm_kernel(x_ref, w_ref, wa_ref, o_ref, o2_ref, *, relu_in):
    xv = x_ref[...]
    if relu_in:
        xv = jnp.maximum(xv, 0.0)
    o_ref[...] = jnp.dot(xv, w_ref[...], preferred_element_type=jnp.float32)
    o2_ref[...] = lax.dot_general(wa_ref[...], xv, (((0,), (1,)), ((), ())),
                                  preferred_element_type=jnp.float32)


def _matmul(xin, w_aug, wa_cols, relu_in):
    m, k = xin.shape
    kd = w_aug.shape[0]
    assert k == kd
    grid = (m + MB - 1) // MB
    return pl.pallas_call(
        functools.partial(_mm_kernel, relu_in=relu_in),
        grid=(grid,),
        in_specs=[
            pl.BlockSpec((MB, k), lambda i: (i, 0)),
            pl.BlockSpec((k, DP), lambda i: (0, 0)),
            pl.BlockSpec((k, AW), lambda i: (0, 0)),
        ],
        out_specs=[
            pl.BlockSpec((MB, DP), lambda i: (i, 0)),
            pl.BlockSpec((AW, MB), lambda i: (0, i)),
        ],
        out_shape=[
            jax.ShapeDtypeStruct((m, DP), jnp.float32),
            jax.ShapeDtypeStruct((AW, NPO), jnp.float32),
        ],
    )(xin, w_aug, wa_cols)


def _gat_sc_body(ow, xp_hbm, alpha_hbm, src_hbm, off_hbm, b_hbm, out_hbm,
                 srcids_v, w_v, asrc_v, adst_v, off_v, acc0_v, acc1_v, row_v, bias_v,
                 stage_v, sem0, sem1, semf0, semf1):
    sems = (sem0, sem1)
    semf = (semf0, semf1)
    accs = (acc0_v, acc1_v)
    wid = lax.axis_index("s") * NC + lax.axis_index("c")
    lo = wid * NPT
    cnt = jnp.minimum(NPT, N - lo)

    pltpu.sync_copy(alpha_hbm.at[pl.ds(0, NP_PAD)], asrc_v)
    pltpu.sync_copy(alpha_hbm.at[pl.ds(8 * NPO + lo, NPT)],
                    adst_v.at[pl.ds(0, NPT)])
    pltpu.sync_copy(off_hbm.at[pl.ds(lo, OFF_SL)], off_v)
    pltpu.sync_copy(b_hbm, bias_v)

    e_base = _sget(off_v, 0)
    ecnt = _sget(off_v, cnt) - e_base
    ebase_al = (e_base // 8) * 8
    sh = e_base - ebase_al

    nch = jnp.minimum((sh + ecnt + 24 + CC - 1) // CC, SRC_CHUNKS)

    def copy_chunk(c, _):
        pltpu.sync_copy(src_hbm.at[pl.ds(ebase_al + c * CC, CC)],
                        srcids_v.at[pl.ds(c * CC, CC)])
        return 0

    lax.fori_loop(0, nch, copy_chunk, 0)

    lane = lax.iota(jnp.int32, 16)

    def out_row_ref(j):
        if ow == DP:
            return out_hbm.at[lo + j]
        return out_hbm.at[pl.ds((lo + j) * ow, ow)]

    def flush_wait(ab, j):
        pltpu.make_async_copy(accs[ab].at[pl.ds(0, ow)], out_row_ref(j),
                              semf[ab]).wait()

    def node_work(j, ab):
        s_g = _sget(off_v, j)
        n_e = _sget(off_v, j + 1) - s_g
        sl = s_g - ebase_al
        adj = _sget(adst_v, j)
        nc16 = (n_e + 15) // 16

        def max_body(c, m):
            idx = srcids_v[pl.ds(sl + c * 16, 16)]
            av = plsc.load_gather(asrc_v, [idx])
            ev = av + adj
            ev = jnp.where(ev >= 0.0, ev, 0.2 * ev)
            w_v[pl.ds(sl + c * 16, 16)] = ev
            valid = (c * 16 + lane) < n_e
            return jnp.maximum(m, jnp.max(jnp.where(valid, ev, NEG)))

        emax = lax.fori_loop(0, nc16, max_body, NEG)

        def sum_body(c, d):
            ev = w_v[pl.ds(sl + c * 16, 16)]
            ee = jnp.exp(ev - emax)
            w_v[pl.ds(sl + c * 16, 16)] = ee
            valid = (c * 16 + lane) < n_e
            return d + jnp.where(valid, ee, 0.0)

        dvec = lax.fori_loop(0, nc16, sum_body, jnp.zeros((16,), jnp.float32))
        rdv = 1.0 / jnp.full((16,), jnp.sum(dvec), jnp.float32)

        nb = (n_e + GB - 1) // GB

        def issue(k, buf):
            iv = srcids_v[pl.ds(sl + k * GB, 16)]
            stage_v[buf, :] = iv
            pltpu.async_copy(xp_hbm.at[stage_v.at[buf].at[pl.ds(0, GB)]],
                             row_v.at[buf], sems[buf])

        def wait(buf):
            pltpu.make_async_copy(xp_hbm.at[stage_v.at[buf].at[pl.ds(0, GB)]],
                                  row_v.at[buf], sems[buf]).wait()

        def compute(k, buf, init):
            ws = []
            for r in range(GB):
                g = k * GB + r
                ws.append(jnp.where(g < n_e, _sget(w_v, sl + g), 0.0))

            def cbody(c, _):
                dsl = pl.ds(c * 16, 16)
                if init:
                    t = ws[0] * row_v[buf, 0, dsl]
                    rr = range(1, GB)
                else:
                    t = accs[ab][dsl]
                    rr = range(GB)
                for r in rr:
                    t = t + ws[r] * row_v[buf, r, dsl]
                accs[ab][dsl] = t
                return 0

            lax.fori_loop(0, DP // 16, cbody, 0)

        issue(0, 0)

        def pair_body(kk, _):
            k0 = kk * 2
            k1 = k0 + 1

            @pl.when(k1 < nb)
            def _():
                issue(k1, 1)

            wait(0)

            @pl.when(k0 == 0)
            def _():
                compute(0, 0, True)

            @pl.when(k0 > 0)
            def _():
                compute(k0, 0, False)

            @pl.when(k1 < nb)
            def _():
                @pl.when(k1 + 1 < nb)
                def _():
                    issue(k1 + 1, 0)

                wait(1)
                compute(k1, 1, False)

            return 0

        lax.fori_loop(0, (nb + 1) // 2, pair_body, 0)

        def flush_body(c, _):
            dsl = pl.ds(c * 16, 16)
            accs[ab][dsl] = accs[ab][dsl] * rdv + bias_v[dsl]
            return 0

        lax.fori_loop(0, DP // 16, flush_body, 0)

        pltpu.async_copy(accs[ab].at[pl.ds(0, ow)], out_row_ref(j),
                         semf[ab]).wait()

    def node_pair(jj, _):
        j0 = jj * 2
        j1 = j0 + 1

        node_work(j0, 0)

        @pl.when(j1 < cnt)
        def _():
            node_work(j1, 1)

        return 0

    lax.fori_loop(0, (cnt + 1) // 2, node_pair, 0)


def _gat_sc(xp, alpha2, src_p, off_p, b_p, ow):
    mesh = plsc.VectorSubcoreMesh(core_axis_name="c", subcore_axis_name="s",
                                  num_cores=NC, num_subcores=NS)
    oshape = (N, DP) if ow == DP else (N * ow,)
    f = pl.kernel(
        functools.partial(_gat_sc_body, ow),
        out_type=jax.ShapeDtypeStruct(oshape, jnp.float32),
        mesh=mesh,
        compiler_params=pltpu.CompilerParams(needs_layout_passes=False),
        scratch_types=[
            pltpu.VMEM((SRC_CAP,), jnp.int32),
            pltpu.VMEM((W_CAP,), jnp.float32),
            pltpu.VMEM((NP_PAD,), jnp.float32),
            pltpu.VMEM((NPT + 16,), jnp.float32),
            pltpu.VMEM((OFF_SL,), jnp.int32),
            pltpu.VMEM((DP,), jnp.float32),
            pltpu.VMEM((DP,), jnp.float32),
            pltpu.VMEM((2, GB, DP), jnp.float32),
            pltpu.VMEM((DP,), jnp.float32),
            pltpu.VMEM((2, 16), jnp.int32),
            pltpu.SemaphoreType.DMA,
            pltpu.SemaphoreType.DMA,
            pltpu.SemaphoreType.DMA,
            pltpu.SemaphoreType.DMA,
        ],
    )
    return f(xp, alpha2, src_p, off_p, b_p)


def _augment(w, a_s, a_d):
    k = w.shape[0]
    wa = jnp.zeros((k, DP), jnp.float32)
    wa = wa.at[:, :D].set(w)
    wa = wa.at[:, D].set(w @ a_s)
    wa = wa.at[:, D + 1].set(w @ a_d)
    return wa


def kernel(x, edge_index, W1, a_src1, a_dst1, b1, W2, a_src2, a_dst2, b2):
    loop = jnp.arange(N, dtype=edge_index.dtype)
    src = jnp.concatenate([edge_index[0], loop])
    dst = jnp.concatenate([edge_index[1], loop])
    perm = jnp.argsort(dst)
    dst_s = dst[perm]
    src_s = src[perm]
    offsets = jnp.searchsorted(dst_s, jnp.arange(N + 1),
                               side="left").astype(jnp.int32)

    src_p = jnp.zeros((EPP,), jnp.int32).at[:EP].set(src_s)
    off_p = jnp.full((OFFP,), EP, jnp.int32).at[:N + 1].set(offsets)

    w1a = _augment(W1, a_src1, a_dst1)
    w2a = jnp.zeros((DP, DP), jnp.float32).at[:D].set(
        _augment(W2, a_src2, a_dst2))

    def layer(xin, w_aug, b, relu_in, ow):
        k = w_aug.shape[0]
        wa_cols = (jnp.zeros((k, AW), jnp.float32)
                   .at[:, 0].set(w_aug[:, D])
                   .at[:, 8].set(w_aug[:, D + 1]))
        xp, alpha2 = _matmul(xin, w_aug, wa_cols, relu_in)
        b_p = jnp.zeros((DP,), jnp.float32).at[:D].set(b)
        return _gat_sc(xp, alpha2.reshape(AW * NPO), src_p, off_p, b_p, ow)

    h = layer(x, w1a, b1, False, DP)
    return layer(h, w2a, b2, True, D).reshape(N, D)


# speedup vs baseline: 1.1397x; 1.1397x over previous
import functools

import jax
import jax.numpy as jnp
from jax import lax
from jax.experimental import pallas as pl
from jax.experimental.pallas import tpu as pltpu
from jax.experimental.pallas import tpu_sc as plsc

N = 10000
D = 2000
E = 40000
EP = E + N
DP = 2048

NC = 2
NS = 16
NW = NC * NS
NPT = 320
OFF_SL = 336
NP_PAD = 10240
OFFP = 10264
CC = 2048
SRC_CHUNKS = 20
SRC_CAP = SRC_CHUNKS * CC
W_CAP = 40368
EPP = 93184
GB = 4
NEG = -3.4e38

MB = 512
AW = 128
NPO = 10240


def _sget(ref, i):
    return ref[pl.ds(i, 16)][0]


def _mm_kernel(x_ref, w_ref, wa_ref, o_ref, o2_ref, *, relu_in):
    xv = x_ref[...]
    if relu_in:
        xv = jnp.maximum(xv, 0.0)
    o_ref[...] = jnp.dot(xv, w_ref[...], preferred_element_type=jnp.float32)
    o2_ref[...] = lax.dot_general(wa_ref[...], xv, (((0,), (1,)), ((), ())),
                                  preferred_element_type=jnp.float32)


def _matmul(xin, w_aug, wa_cols, relu_in):
    m, k = xin.shape
    kd = w_aug.shape[0]
    assert k == kd
    grid = (m + MB - 1) // MB
    return pl.pallas_call(
        functools.partial(_mm_kernel, relu_in=relu_in),
        grid=(grid,),
        in_specs=[
            pl.BlockSpec((MB, k), lambda i: (i, 0)),
            pl.BlockSpec((k, DP), lambda i: (0, 0)),
            pl.BlockSpec((k, AW), lambda i: (0, 0)),
        ],
        out_specs=[
            pl.BlockSpec((MB, DP), lambda i: (i, 0)),
            pl.BlockSpec((AW, MB), lambda i: (0, i)),
        ],
        out_shape=[
            jax.ShapeDtypeStruct((m, DP), jnp.float32),
            jax.ShapeDtypeStruct((AW, NPO), jnp.float32),
        ],
    )(xin, w_aug, wa_cols)


def _gat_sc_body(ow, xp_hbm, alpha_hbm, src_hbm, off_hbm, b_hbm, out_hbm,
                 srcids_v, w_v, asrc_v, adst_v, off_v, acc0_v, acc1_v, fbuf0_v,
                 fbuf1_v, row_v, bias_v,
                 stage_v, sem0, sem1, semf0, semf1):
    sems = (sem0, sem1)
    semf = (semf0, semf1)
    accs = (acc0_v, acc1_v)
    fbufs = (fbuf0_v, fbuf1_v)
    wid = lax.axis_index("s") * NC + lax.axis_index("c")
    lo = wid * NPT
    cnt = jnp.minimum(NPT, N - lo)

    pltpu.sync_copy(alpha_hbm.at[pl.ds(0, NP_PAD)], asrc_v)
    pltpu.sync_copy(alpha_hbm.at[pl.ds(8 * NPO + lo, NPT)],
                    adst_v.at[pl.ds(0, NPT)])
    pltpu.sync_copy(off_hbm.at[pl.ds(lo, OFF_SL)], off_v)
    pltpu.sync_copy(b_hbm, bias_v)

    e_base = _sget(off_v, 0)
    ecnt = _sget(off_v, cnt) - e_base
    ebase_al = (e_base // 8) * 8
    sh = e_base - ebase_al

    nch = jnp.minimum((sh + ecnt + 24 + CC - 1) // CC, SRC_CHUNKS)

    def copy_chunk(c, _):
        pltpu.sync_copy(src_hbm.at[pl.ds(ebase_al + c * CC, CC)],
                        srcids_v.at[pl.ds(c * CC, CC)])
        return 0

    lax.fori_loop(0, nch, copy_chunk, 0)

    lane = lax.iota(jnp.int32, 16)

    def out_row_ref(j):
        if ow == DP:
            return out_hbm.at[lo + j]
        return out_hbm.at[pl.ds((lo + j) * ow, ow)]

    def flush_wait(ab, j):
        pltpu.make_async_copy(accs[ab].at[pl.ds(0, ow)], out_row_ref(j),
                              semf[ab]).wait()

    def flush_wait(ab):
        pltpu.make_async_copy(fbufs[ab].at[pl.ds(0, ow)], out_row_ref(0),
                              semf[ab]).wait()

    def node_work(j, ab, wait_pred):
        s_g = _sget(off_v, j)
        n_e = _sget(off_v, j + 1) - s_g
        sl = s_g - ebase_al
        adj = _sget(adst_v, j)
        nc16 = (n_e + 15) // 16

        def max_body(c, m):
            idx = srcids_v[pl.ds(sl + c * 16, 16)]
            av = plsc.load_gather(asrc_v, [idx])
            ev = av + adj
            ev = jnp.where(ev >= 0.0, ev, 0.2 * ev)
            w_v[pl.ds(sl + c * 16, 16)] = ev
            valid = (c * 16 + lane) < n_e
            return jnp.maximum(m, jnp.max(jnp.where(valid, ev, NEG)))

        emax = lax.fori_loop(0, nc16, max_body, NEG)

        def sum_body(c, d):
            ev = w_v[pl.ds(sl + c * 16, 16)]
            ee = jnp.exp(ev - emax)
            w_v[pl.ds(sl + c * 16, 16)] = ee
            valid = (c * 16 + lane) < n_e
            return d + jnp.where(valid, ee, 0.0)

        dvec = lax.fori_loop(0, nc16, sum_body, jnp.zeros((16,), jnp.float32))
        rdv = 1.0 / jnp.full((16,), jnp.sum(dvec), jnp.float32)

        nb = (n_e + GB - 1) // GB

        def issue(k, buf):
            iv = srcids_v[pl.ds(sl + k * GB, 16)]
            stage_v[buf, :] = iv
            pltpu.async_copy(xp_hbm.at[stage_v.at[buf].at[pl.ds(0, GB)]],
                             row_v.at[buf], sems[buf])

        def wait(buf):
            pltpu.make_async_copy(xp_hbm.at[stage_v.at[buf].at[pl.ds(0, GB)]],
                                  row_v.at[buf], sems[buf]).wait()

        def compute(k, buf, wslot, init):
            ws = []
            for r in range(GB):
                g = k * GB + r
                ws.append(jnp.where(g < n_e, _sget(w_v, sl + g), 0.0))

            def cbody(c, _):
                for u in range(4):
                    dsl = pl.ds((c * 4 + u) * 16, 16)
                    if init:
                        t = ws[0] * row_v[buf, 0, dsl]
                        rr = range(1, GB)
                    else:
                        t = accs[1 - wslot][dsl]
                        rr = range(GB)
                    for r in rr:
                        t = t + ws[r] * row_v[buf, r, dsl]
                    accs[wslot][dsl] = t
                return 0

            lax.fori_loop(0, DP // 64, cbody, 0)

        issue(0, 0)

        def pair_body(kk, _):
            k0 = kk * 2
            k1 = k0 + 1

            @pl.when(k1 < nb)
            def _():
                issue(k1, 1)

            wait(0)

            @pl.when(k0 == 0)
            def _():
                compute(0, 0, 0, True)

            @pl.when(k0 > 0)
            def _():
                compute(k0, 0, 0, False)

            @pl.when(k1 < nb)
            def _():
                @pl.when(k1 + 1 < nb)
                def _():
                    issue(k1 + 1, 0)

                wait(1)
                compute(k1, 1, 1, False)

            return 0

        lax.fori_loop(0, (nb + 1) // 2, pair_body, 0)

        @pl.when(wait_pred)
        def _():
            flush_wait(ab)

        par = (nb - 1) % 2

        for p in range(2):
            @pl.when(par == p)
            def _(p=p):
                def flush_body(c, _):
                    for u in range(4):
                        dsl = pl.ds((c * 4 + u) * 16, 16)
                        fbufs[ab][dsl] = accs[p][dsl] * rdv + bias_v[dsl]
                    return 0

                lax.fori_loop(0, DP // 64, flush_body, 0)

        pltpu.async_copy(fbufs[ab].at[pl.ds(0, ow)], out_row_ref(j),
                         semf[ab])

    def node_pair(jj, _):
        j0 = jj * 2
        j1 = j0 + 1

        node_work(j0, 0, j0 > 0)

        @pl.when(j1 < cnt)
        def _():
            node_work(j1, 1, j1 > 1)

        return 0

    lax.fori_loop(0, (cnt + 1) // 2, node_pair, 0)
    flush_wait(0)

    @pl.when(cnt > 1)
    def _():
        flush_wait(1)


def _gat_sc(xp, alpha2, src_p, off_p, b_p, ow):
    mesh = plsc.VectorSubcoreMesh(core_axis_name="c", subcore_axis_name="s",
                                  num_cores=NC, num_subcores=NS)
    oshape = (N, DP) if ow == DP else (N * ow,)
    f = pl.kernel(
        functools.partial(_gat_sc_body, ow),
        out_type=jax.ShapeDtypeStruct(oshape, jnp.float32),
        mesh=mesh,
        compiler_params=pltpu.CompilerParams(needs_layout_passes=False),
        scratch_types=[
            pltpu.VMEM((SRC_CAP,), jnp.int32),
            pltpu.VMEM((W_CAP,), jnp.float32),
            pltpu.VMEM((NP_PAD,), jnp.float32),
            pltpu.VMEM((NPT + 16,), jnp.float32),
            pltpu.VMEM((OFF_SL,), jnp.int32),
            pltpu.VMEM((DP,), jnp.float32),
            pltpu.VMEM((DP,), jnp.float32),
            pltpu.VMEM((DP,), jnp.float32),
            pltpu.VMEM((DP,), jnp.float32),
            pltpu.VMEM((2, GB, DP), jnp.float32),
            pltpu.VMEM((DP,), jnp.float32),
            pltpu.VMEM((2, 16), jnp.int32),
            pltpu.SemaphoreType.DMA,
            pltpu.SemaphoreType.DMA,
            pltpu.SemaphoreType.DMA,
            pltpu.SemaphoreType.DMA,
        ],
    )
    return f(xp, alpha2, src_p, off_p, b_p)


def _augment(w, a_s, a_d):
    k = w.shape[0]
    wa = jnp.zeros((k, DP), jnp.float32)
    wa = wa.at[:, :D].set(w)
    wa = wa.at[:, D].set(w @ a_s)
    wa = wa.at[:, D + 1].set(w @ a_d)
    return wa


def kernel(x, edge_index, W1, a_src1, a_dst1, b1, W2, a_src2, a_dst2, b2):
    loop = jnp.arange(N, dtype=edge_index.dtype)
    src = jnp.concatenate([edge_index[0], loop])
    dst = jnp.concatenate([edge_index[1], loop])
    perm = jnp.argsort(dst)
    dst_s = dst[perm]
    src_s = src[perm]
    offsets = jnp.searchsorted(dst_s, jnp.arange(N + 1),
                               side="left").astype(jnp.int32)

    src_p = jnp.zeros((EPP,), jnp.int32).at[:EP].set(src_s)
    off_p = jnp.full((OFFP,), EP, jnp.int32).at[:N + 1].set(offsets)

    w1a = _augment(W1, a_src1, a_dst1)
    w2a = jnp.zeros((DP, DP), jnp.float32).at[:D].set(
        _augment(W2, a_src2, a_dst2))

    def layer(xin, w_aug, b, relu_in, ow):
        k = w_aug.shape[0]
        wa_cols = (jnp.zeros((k, AW), jnp.float32)
                   .at[:, 0].set(w_aug[:, D])
                   .at[:, 8].set(w_aug[:, D + 1]))
        xp, alpha2 = _matmul(xin, w_aug, wa_cols, relu_in)
        b_p = jnp.zeros((DP,), jnp.float32).at[:D].set(b)
        return _gat_sc(xp, alpha2.reshape(AW * NPO), src_p, off_p, b_p, ow)

    h = layer(x, w1a, b1, False, DP)
    return layer(h, w2a, b2, True, D).reshape(N, D)
